# baseline (speedup 1.0000x reference)
import jax
import jax.numpy as jnp
from jax.experimental import pallas as pl
from jax.experimental.pallas import tpu as pltpu

_B, _L, _LOC_MAX, _EMB = 4, 50, 2000, 16
_SU, _SL, _TU, _TL = 100.0, 0.0, 500.0, 0.0
_G = 25
_P = (_B * _L) // _G


def _body(idx_ref, vf_ref, vecv_ref, esl_ref, esu_ref, etl_ref, etu_ref,
          mat2_ref, out_ref, rows_buf, gsems):
    g = pl.program_id(0)

    def gissue(gg, slot):
        for i in range(_P):
            pltpu.make_async_copy(
                mat2_ref.at[idx_ref[gg * _P + i]],
                rows_buf.at[slot, i],
                gsems.at[slot],
            ).start()

    @pl.when(g == 0)
    def _():
        gissue(g, g % 2)

    @pl.when(g + 1 < _G)
    def _():
        gissue(g + 1, (g + 1) % 2)

    slot = g % 2
    for i in range(_P):
        pltpu.make_async_copy(
            mat2_ref.at[idx_ref[g * _P + i]],
            rows_buf.at[slot, i],
            gsems.at[slot],
        ).wait()

    v = vf_ref[0]
    t = vecv_ref[0]

    def sel(ref):
        lo = ref[0:1, :]
        return lo + v * (ref[1:2, :] - lo)

    esl = sel(esl_ref)
    esu = sel(esu_ref)
    etl = sel(etl_ref)
    etu = sel(etu_ref)
    base = esl + etl + (etu - etl) * (t * (1.0 / _TU))
    coef = (esu - esl) * (v * (1.0 / _SU))

    rows = rows_buf[slot].reshape(_P, _LOC_MAX)
    rows_t = jnp.transpose(rows)
    for i in range(_P):
        col = rows_t[:, i:i + 1]
        out_ref[0, i] = col * coef[i:i + 1, :] + base[i:i + 1, :]


def kernel(traj_loc, mat2, vec, traj_len, emb_su, emb_sl, emb_tu, emb_tl):
    idx = (traj_loc.reshape(-1) - 1).astype(jnp.int32)
    vf = (jnp.arange(_L)[None, :] < traj_len[:, None]).astype(
        jnp.float32).reshape(_G, _P, 1)
    vecv = vec.astype(jnp.float32).reshape(_G, _P, 1)

    grid_spec = pltpu.PrefetchScalarGridSpec(
        num_scalar_prefetch=1,
        grid=(_G,),
        in_specs=[
            pl.BlockSpec((1, _P, 1), lambda g, i: (g, 0, 0)),
            pl.BlockSpec((1, _P, 1), lambda g, i: (g, 0, 0)),
            pl.BlockSpec((2, _EMB), lambda g, i: (0, 0)),
            pl.BlockSpec((2, _EMB), lambda g, i: (0, 0)),
            pl.BlockSpec((2, _EMB), lambda g, i: (0, 0)),
            pl.BlockSpec((2, _EMB), lambda g, i: (0, 0)),
            pl.BlockSpec(memory_space=pl.ANY),
        ],
        out_specs=pl.BlockSpec(
            (1, _P, _LOC_MAX, _EMB), lambda g, i: (g, 0, 0, 0)),
        scratch_shapes=[
            pltpu.VMEM((2, _P, 1, _LOC_MAX), jnp.float32),
            pltpu.SemaphoreType.DMA((2,)),
        ],
    )
    out = pl.pallas_call(
        _body,
        grid_spec=grid_spec,
        out_shape=jax.ShapeDtypeStruct((_G, _P, _LOC_MAX, _EMB), jnp.float32),
    )(idx, vf, vecv, emb_sl, emb_su, emb_tl, emb_tu,
      mat2.reshape(_LOC_MAX, 1, _LOC_MAX))
    return out.reshape(_B, _L, _LOC_MAX, _EMB)


# speedup vs baseline: 1.0252x; 1.0252x over previous
import jax
import jax.numpy as jnp
from jax.experimental import pallas as pl
from jax.experimental.pallas import tpu as pltpu

_B, _L, _LOC_MAX, _EMB = 4, 50, 2000, 16
_SU, _SL, _TU, _TL = 100.0, 0.0, 500.0, 0.0
_G = 10
_P = (_B * _L) // _G


def _body(idx_ref, vf_ref, vecv_ref, esl_ref, esu_ref, etl_ref, etu_ref,
          mat2_ref, out_ref, rows_buf, gsems):
    g = pl.program_id(0)

    def gissue(gg, slot):
        for i in range(_P):
            pltpu.make_async_copy(
                mat2_ref.at[idx_ref[gg * _P + i]],
                rows_buf.at[slot, i],
                gsems.at[slot],
            ).start()

    @pl.when(g == 0)
    def _():
        gissue(g, g % 2)

    @pl.when(g + 1 < _G)
    def _():
        gissue(g + 1, (g + 1) % 2)

    slot = g % 2
    for i in range(_P):
        pltpu.make_async_copy(
            mat2_ref.at[idx_ref[g * _P + i]],
            rows_buf.at[slot, i],
            gsems.at[slot],
        ).wait()

    v = vf_ref[0]
    t = vecv_ref[0]

    def sel(ref):
        lo = ref[0:1, :]
        return lo + v * (ref[1:2, :] - lo)

    esl = sel(esl_ref)
    esu = sel(esu_ref)
    etl = sel(etl_ref)
    etu = sel(etu_ref)
    base = esl + etl + (etu - etl) * (t * (1.0 / _TU))
    coef = (esu - esl) * (v * (1.0 / _SU))

    rows = rows_buf[slot].reshape(_P, _LOC_MAX)
    rows_t = jnp.transpose(rows)
    for i in range(_P):
        col = rows_t[:, i:i + 1]
        out_ref[0, i] = col * coef[i:i + 1, :] + base[i:i + 1, :]


def kernel(traj_loc, mat2, vec, traj_len, emb_su, emb_sl, emb_tu, emb_tl):
    idx = (traj_loc.reshape(-1) - 1).astype(jnp.int32)
    vf = (jnp.arange(_L)[None, :] < traj_len[:, None]).astype(
        jnp.float32).reshape(_G, _P, 1)
    vecv = vec.astype(jnp.float32).reshape(_G, _P, 1)

    grid_spec = pltpu.PrefetchScalarGridSpec(
        num_scalar_prefetch=1,
        grid=(_G,),
        in_specs=[
            pl.BlockSpec((1, _P, 1), lambda g, i: (g, 0, 0)),
            pl.BlockSpec((1, _P, 1), lambda g, i: (g, 0, 0)),
            pl.BlockSpec((2, _EMB), lambda g, i: (0, 0)),
            pl.BlockSpec((2, _EMB), lambda g, i: (0, 0)),
            pl.BlockSpec((2, _EMB), lambda g, i: (0, 0)),
            pl.BlockSpec((2, _EMB), lambda g, i: (0, 0)),
            pl.BlockSpec(memory_space=pl.ANY),
        ],
        out_specs=pl.BlockSpec(
            (1, _P, _LOC_MAX, _EMB), lambda g, i: (g, 0, 0, 0)),
        scratch_shapes=[
            pltpu.VMEM((2, _P, 1, _LOC_MAX), jnp.float32),
            pltpu.SemaphoreType.DMA((2,)),
        ],
    )
    out = pl.pallas_call(
        _body,
        grid_spec=grid_spec,
        out_shape=jax.ShapeDtypeStruct((_G, _P, _LOC_MAX, _EMB), jnp.float32),
    )(idx, vf, vecv, emb_sl, emb_su, emb_tl, emb_tu,
      mat2.reshape(_LOC_MAX, 1, _LOC_MAX))
    return out.reshape(_B, _L, _LOC_MAX, _EMB)
